# Initial kernel scaffold; baseline (speedup 1.0000x reference)
import jax
import jax.numpy as jnp
from jax.experimental import pallas as pl


def kernel(positions, directions, embeddings, W1, b1, W2, b2, Wc1, bc1, Wc2, bc2, Wc3, bc3):
    raise NotImplementedError("write your pallas kernel here")


# speedup vs baseline: 28.3531x; 28.3531x over previous
import functools

import jax
import jax.numpy as jnp
import numpy as np
from jax import lax
from jax.experimental import pallas as pl
from jax.experimental.pallas import tpu as pltpu
from jax.experimental.pallas import tpu_sc as plsc

NUM_LEVELS = 16
BASE_RES = 16
FINEST = 512
LOG2_HASH = 19
FEAT = 2
HASHMAP = 2 ** LOG2_HASH
RES = []
OFF = []
_total = 0
for _lv in range(NUM_LEVELS):
    _r = min(int(BASE_RES * (2.0 ** _lv)), FINEST)
    RES.append(_r)
    OFF.append(_total)
    _total += min(_r ** 3, HASHMAP)
TOTAL_PARAMS = _total
MASKS = [min(r ** 3, HASHMAP) - 1 for r in RES]

HX = np.int32(np.uint32(73856093) & 0xFFFFFFFF)
HY = np.int32(19349663)
HZ = np.int32(np.uint32(83492791) & 0xFFFFFFFF)
I32 = np.int32

NW = 32
C = 128
K = 8 * C
G = C // 16


def _sc_encode(pos_t, emb0, emb1, rm1f, rm1i, maskv, offv):
    N = pos_t.shape[1]
    NP = N // NW
    NCHUNK = NP // C
    mesh = plsc.VectorSubcoreMesh(core_axis_name="c", subcore_axis_name="s")

    @functools.partial(
        pl.kernel,
        mesh=mesh,
        out_type=jax.ShapeDtypeStruct((2 * NUM_LEVELS, N), jnp.float32),
        scratch_types=[
            pltpu.VMEM((3, C), jnp.float32),
            pltpu.VMEM((8 * NUM_LEVELS, C), jnp.int32),
            pltpu.VMEM((3 * NUM_LEVELS, C), jnp.float32),
            pltpu.VMEM((8 * NUM_LEVELS, C), jnp.float32),
            pltpu.VMEM((8 * NUM_LEVELS, C), jnp.float32),
            pltpu.VMEM((2 * NUM_LEVELS, C), jnp.float32),
            pltpu.VMEM((16,), jnp.float32),
            pltpu.VMEM((16,), jnp.int32),
            pltpu.VMEM((16,), jnp.int32),
            pltpu.VMEM((16,), jnp.int32),
            pltpu.SemaphoreType.DMA,
        ],
        compiler_params=pltpu.CompilerParams(needs_layout_passes=False),
    )
    def enc_kernel(pos_hbm, emb0_hbm, emb1_hbm, rm1f_hbm, rm1i_hbm, mask_hbm,
                   off_hbm, enc_hbm, pos_v, idx_v, w3_v, rows0_v, rows1_v,
                   acc_v, rm1f_v, rm1i_v, mask_v, off_v, sem):
        wid = lax.axis_index("s") * 2 + lax.axis_index("c")
        base0 = wid * I32(NP)
        pltpu.sync_copy(rm1f_hbm, rm1f_v)
        pltpu.sync_copy(rm1i_hbm, rm1i_v)
        pltpu.sync_copy(mask_hbm, mask_v)
        pltpu.sync_copy(off_hbm, off_v)
        iota = lax.iota(jnp.int32, 16)

        def chunk_body(chunk, _):
            base = base0 + chunk * I32(C)
            pltpu.sync_copy(pos_hbm.at[:, pl.ds(base, C)], pos_v)

            def norm_body(g, _):
                gb = g * I32(16)
                for d in range(3):
                    p = pos_v[d, pl.ds(gb, 16)]
                    p01 = jnp.minimum(jnp.maximum((p + 1.0) * 0.5, 0.0), 1.0)
                    pos_v[d, pl.ds(gb, 16)] = p01
                return 0
            lax.fori_loop(I32(0), I32(G), norm_body, 0, unroll=True)

            def levelA(lv, _):
                lvv = jnp.full((16,), lv, jnp.int32)
                rm1f_b = plsc.load_gather(rm1f_v, [lvv])
                rm1i_b = plsc.load_gather(rm1i_v, [lvv])
                mask_b = plsc.load_gather(mask_v, [lvv])
                off_b = plsc.load_gather(off_v, [lvv])

                def groupA(g, _):
                    gb = g * I32(16)
                    px = pos_v[0, pl.ds(gb, 16)]
                    py = pos_v[1, pl.ds(gb, 16)]
                    pz = pos_v[2, pl.ds(gb, 16)]
                    psx = px * rm1f_b
                    psy = py * rm1f_b
                    psz = pz * rm1f_b
                    pgx = psx.astype(jnp.int32)
                    pgy = psy.astype(jnp.int32)
                    pgz = psz.astype(jnp.int32)
                    w3_v[I32(3) * lv + I32(0), pl.ds(gb, 16)] = psx - pgx.astype(jnp.float32)
                    w3_v[I32(3) * lv + I32(1), pl.ds(gb, 16)] = psy - pgy.astype(jnp.float32)
                    w3_v[I32(3) * lv + I32(2), pl.ds(gb, 16)] = psz - pgz.astype(jnp.float32)
                    x0 = pgx * HX
                    x1 = jnp.minimum(pgx + I32(1), rm1i_b) * HX
                    y0 = pgy * HY
                    y1 = jnp.minimum(pgy + I32(1), rm1i_b) * HY
                    z0 = pgz * HZ
                    z1 = jnp.minimum(pgz + I32(1), rm1i_b) * HZ
                    xy = (x0 + y0, x0 + y1, x1 + y0, x1 + y1)
                    zz = (z0, z1)
                    for j in range(8):
                        h = (xy[j >> 1] + zz[j & 1]) & mask_b
                        idx_v[I32(8) * lv + I32(j), pl.ds(gb, 16)] = h + off_b
                    return 0
                lax.fori_loop(I32(0), I32(G), groupA, 0, unroll=True)
                for j in range(8):
                    r = I32(8) * lv + I32(j)
                    pltpu.async_copy(emb0_hbm.at[idx_v.at[r]], rows0_v.at[r], sem)
                    pltpu.async_copy(emb1_hbm.at[idx_v.at[r]], rows1_v.at[r], sem)
                return 0
            lax.fori_loop(I32(0), I32(NUM_LEVELS), levelA, 0)

            def drain(r, _):
                pltpu.make_async_copy(
                    emb0_hbm.at[idx_v.at[r]], rows0_v.at[r], sem).wait()
                pltpu.make_async_copy(
                    emb1_hbm.at[idx_v.at[r]], rows1_v.at[r], sem).wait()
                return 0
            lax.fori_loop(I32(0), I32(8 * NUM_LEVELS), drain, 0)

            def levelB(lv, _):
                def groupB(g, _):
                    gb = g * I32(16)
                    wx = w3_v[I32(3) * lv + I32(0), pl.ds(gb, 16)]
                    wy = w3_v[I32(3) * lv + I32(1), pl.ds(gb, 16)]
                    wz = w3_v[I32(3) * lv + I32(2), pl.ds(gb, 16)]
                    for f, rows in ((0, rows0_v), (1, rows1_v)):
                        e = [rows[I32(8) * lv + I32(j), pl.ds(gb, 16)]
                             for j in range(8)]
                        a00 = e[0] + wx * (e[4] - e[0])
                        a01 = e[1] + wx * (e[5] - e[1])
                        a10 = e[2] + wx * (e[6] - e[2])
                        a11 = e[3] + wx * (e[7] - e[3])
                        b0 = a00 + wy * (a10 - a00)
                        b1 = a01 + wy * (a11 - a01)
                        acc_v[I32(2) * lv + I32(f), pl.ds(gb, 16)] = b0 + wz * (b1 - b0)
                    return 0
                lax.fori_loop(I32(0), I32(G), groupB, 0, unroll=True)
                return 0
            lax.fori_loop(I32(0), I32(NUM_LEVELS), levelB, 0)

            pltpu.sync_copy(acc_v, enc_hbm.at[:, pl.ds(base, C)])
            return 0

        lax.fori_loop(I32(0), I32(NCHUNK), chunk_body, 0)

    return enc_kernel(pos_t, emb0, emb1, rm1f, rm1i, maskv, offv)


def _tc_mlp(enc_t, dirs_t, W1T, b1, W2T, b2, Wc1T, bc1, Wc2T, bc2, Wc3T, bc3):
    N = enc_t.shape[1]
    B = 2048
    HI = jax.lax.Precision.HIGHEST

    def body(enc_ref, dirs_ref, W1_ref, b1_ref, W2_ref, b2_ref,
             Wc1_ref, bc1_ref, Wc2_ref, bc2_ref, Wc3_ref, bc3_ref, out_ref):
        enc = enc_ref[...]
        h1 = jnp.maximum(
            jnp.dot(W1_ref[...], enc, precision=HI) + b1_ref[...], 0.0)
        h2 = jnp.dot(W2_ref[...], h1, precision=HI) + b2_ref[...]
        sigma = jnp.exp(h2[0:1, :])
        geo = h2[1:16, :]
        d = dirs_ref[...]
        x = d[0:1, :]
        y = d[1:2, :]
        z = d[2:3, :]
        norm = jnp.sqrt(x * x + y * y + z * z)
        x = x / norm
        y = y / norm
        z = z / norm
        sh = jnp.concatenate([
            jnp.full_like(x, 0.28209479177387814),
            -0.48860251190291987 * y,
            0.48860251190291987 * z,
            -0.48860251190291987 * x,
            1.0925484305920792 * x * y,
            -1.0925484305920792 * y * z,
            0.31539156525252005 * (2 * z * z - x * x - y * y),
            -1.0925484305920792 * x * z,
            0.5462742152960396 * (x * x - y * y),
            -0.5900435899266435 * y * (3 * x * x - y * y),
            2.890611442640554 * x * y * z,
            -0.4570457994644658 * y * (4 * z * z - x * x - y * y),
            0.3731763325901154 * z * (2 * z * z - 3 * x * x - 3 * y * y),
            -0.4570457994644658 * x * (4 * z * z - x * x - y * y),
            1.445305721320277 * z * (x * x - y * y),
            -0.5900435899266435 * x * (x * x - 3 * y * y),
        ], axis=0)
        c = jnp.concatenate([sh, geo], axis=0)
        c1 = jnp.maximum(jnp.dot(Wc1_ref[...], c, precision=HI) + bc1_ref[...], 0.0)
        c2 = jnp.maximum(jnp.dot(Wc2_ref[...], c1, precision=HI) + bc2_ref[...], 0.0)
        rgb = jax.nn.sigmoid(jnp.dot(Wc3_ref[...], c2, precision=HI) + bc3_ref[...])
        out_ref[...] = jnp.concatenate([rgb, sigma], axis=0)

    full = lambda shape: pl.BlockSpec(shape, lambda i: (0, 0))
    return pl.pallas_call(
        body,
        grid=(N // B,),
        in_specs=[
            pl.BlockSpec((2 * NUM_LEVELS, B), lambda i: (0, i)),
            pl.BlockSpec((3, B), lambda i: (0, i)),
            full(W1T.shape), full(b1.shape), full(W2T.shape), full(b2.shape),
            full(Wc1T.shape), full(bc1.shape), full(Wc2T.shape), full(bc2.shape),
            full(Wc3T.shape), full(bc3.shape),
        ],
        out_specs=pl.BlockSpec((4, B), lambda i: (0, i)),
        out_shape=jax.ShapeDtypeStruct((4, N), jnp.float32),
    )(enc_t, dirs_t, W1T, b1, W2T, b2, Wc1T, bc1, Wc2T, bc2, Wc3T, bc3)


def kernel(positions, directions, embeddings, W1, b1, W2, b2,
           Wc1, bc1, Wc2, bc2, Wc3, bc3):
    with jax.enable_x64(False):
        out = _kernel_x32(positions, directions, embeddings, W1, b1, W2, b2,
                          Wc1, bc1, Wc2, bc2, Wc3, bc3)
    return out.astype(jnp.float64)


def _kernel_x32(positions, directions, embeddings, W1, b1, W2, b2,
                Wc1, bc1, Wc2, bc2, Wc3, bc3):
    f32 = jnp.float32
    (positions, directions, embeddings, W1, b1, W2, b2,
     Wc1, bc1, Wc2, bc2, Wc3, bc3) = (
        a.astype(f32) for a in (positions, directions, embeddings, W1, b1,
                                W2, b2, Wc1, bc1, Wc2, bc2, Wc3, bc3))
    pos_t = positions.T
    dirs_t = directions.T
    rm1f = jnp.asarray([r - 1 for r in RES], jnp.float32)
    rm1i = jnp.asarray([r - 1 for r in RES], jnp.int32)
    maskv = jnp.asarray(MASKS, jnp.int32)
    offv = jnp.asarray(OFF, jnp.int32)
    emb0 = embeddings[:, 0]
    emb1 = embeddings[:, 1]
    enc_t = _sc_encode(pos_t, emb0, emb1, rm1f, rm1i, maskv, offv)
    out_t = _tc_mlp(
        enc_t, dirs_t,
        W1.T, b1.reshape(-1, 1), W2.T, b2.reshape(-1, 1),
        Wc1.T, bc1.reshape(-1, 1), Wc2.T, bc2.reshape(-1, 1),
        Wc3.T, bc3.reshape(-1, 1),
    )
    return out_t.T


# speedup vs baseline: 44.0844x; 1.5548x over previous
import functools

import jax
import jax.numpy as jnp
import numpy as np
from jax import lax
from jax.experimental import pallas as pl
from jax.experimental.pallas import tpu as pltpu
from jax.experimental.pallas import tpu_sc as plsc

NUM_LEVELS = 16
BASE_RES = 16
FINEST = 512
LOG2_HASH = 19
FEAT = 2
HASHMAP = 2 ** LOG2_HASH
RES = []
OFF = []
_total = 0
for _lv in range(NUM_LEVELS):
    _r = min(int(BASE_RES * (2.0 ** _lv)), FINEST)
    RES.append(_r)
    OFF.append(_total)
    _total += min(_r ** 3, HASHMAP)
TOTAL_PARAMS = _total
MASKS = [min(r ** 3, HASHMAP) - 1 for r in RES]

HX = np.int32(np.uint32(73856093) & 0xFFFFFFFF)
HY = np.int32(19349663)
HZ = np.int32(np.uint32(83492791) & 0xFFFFFFFF)
I32 = np.int32

NW = 32
C = 128
K = 8 * C
G = C // 16


def _sc_encode(pos_t, emb, emb2d, rm1f, rm1i, maskv, offv):
    N = pos_t.shape[1]
    NP = N // NW
    NCHUNK = NP // C
    mesh = plsc.VectorSubcoreMesh(core_axis_name="c", subcore_axis_name="s")

    @functools.partial(
        pl.kernel,
        mesh=mesh,
        out_type=jax.ShapeDtypeStruct((2 * NUM_LEVELS, N), jnp.float32),
        scratch_types=[
            pltpu.VMEM((3, C), jnp.float32),
            pltpu.VMEM((8 * NUM_LEVELS, C), jnp.int32),
            pltpu.VMEM((3 * NUM_LEVELS, C), jnp.float32),
            pltpu.VMEM((8 * NUM_LEVELS, C), jnp.int32),
            pltpu.VMEM((2 * NUM_LEVELS, C), jnp.float32),
            pltpu.VMEM((16,), jnp.float32),
            pltpu.VMEM((16,), jnp.int32),
            pltpu.VMEM((16,), jnp.int32),
            pltpu.VMEM((16,), jnp.int32),
            pltpu.SemaphoreType.DMA,
        ],
        compiler_params=pltpu.CompilerParams(needs_layout_passes=False),
    )
    def enc_kernel(pos_hbm, emb_hbm, emb2d_hbm, rm1f_hbm, rm1i_hbm, mask_hbm,
                   off_hbm, enc_hbm, pos_v, idx_v, w3_v, rows_v,
                   acc_v, rm1f_v, rm1i_v, mask_v, off_v, sem):
        wid = lax.axis_index("s") * 2 + lax.axis_index("c")
        base0 = wid * I32(NP)
        pltpu.sync_copy(rm1f_hbm, rm1f_v)
        pltpu.sync_copy(rm1i_hbm, rm1i_v)
        pltpu.sync_copy(mask_hbm, mask_v)
        pltpu.sync_copy(off_hbm, off_v)
        iota = lax.iota(jnp.int32, 16)

        def chunk_body(chunk, _):
            base = base0 + chunk * I32(C)
            pltpu.sync_copy(pos_hbm.at[:, pl.ds(base, C)], pos_v)

            def norm_body(g, _):
                gb = g * I32(16)
                for d in range(3):
                    p = pos_v[d, pl.ds(gb, 16)]
                    p01 = jnp.minimum(jnp.maximum((p + 1.0) * 0.5, 0.0), 1.0)
                    pos_v[d, pl.ds(gb, 16)] = p01
                return 0
            lax.fori_loop(I32(0), I32(G), norm_body, 0, unroll=True)

            def levelA(lv, _):
                lvv = jnp.full((16,), lv, jnp.int32)
                rm1f_b = plsc.load_gather(rm1f_v, [lvv])
                rm1i_b = plsc.load_gather(rm1i_v, [lvv])
                mask_b = plsc.load_gather(mask_v, [lvv])
                off_b = plsc.load_gather(off_v, [lvv])

                def groupA(g, _):
                    gb = g * I32(16)
                    px = pos_v[0, pl.ds(gb, 16)]
                    py = pos_v[1, pl.ds(gb, 16)]
                    pz = pos_v[2, pl.ds(gb, 16)]
                    psx = px * rm1f_b
                    psy = py * rm1f_b
                    psz = pz * rm1f_b
                    pgx = psx.astype(jnp.int32)
                    pgy = psy.astype(jnp.int32)
                    pgz = psz.astype(jnp.int32)
                    w3_v[I32(3) * lv + I32(0), pl.ds(gb, 16)] = psx - pgx.astype(jnp.float32)
                    w3_v[I32(3) * lv + I32(1), pl.ds(gb, 16)] = psy - pgy.astype(jnp.float32)
                    w3_v[I32(3) * lv + I32(2), pl.ds(gb, 16)] = psz - pgz.astype(jnp.float32)
                    x0 = pgx * HX
                    x1 = jnp.minimum(pgx + I32(1), rm1i_b) * HX
                    y0 = pgy * HY
                    y1 = jnp.minimum(pgy + I32(1), rm1i_b) * HY
                    z0 = pgz * HZ
                    z1 = jnp.minimum(pgz + I32(1), rm1i_b) * HZ
                    xy = (x0 + y0, x0 + y1, x1 + y0, x1 + y1)
                    zz = (z0, z1)
                    for j in range(8):
                        h = (xy[j >> 1] + zz[j & 1]) & mask_b
                        idx_v[I32(8) * lv + I32(j), pl.ds(gb, 16)] = h + off_b
                    return 0
                lax.fori_loop(I32(0), I32(G), groupA, 0, unroll=True)
                for j in range(8):
                    r = I32(8) * lv + I32(j)
                    pltpu.async_copy(emb_hbm.at[idx_v.at[r]], rows_v.at[r], sem)
                return 0
            lax.fori_loop(I32(0), I32(NUM_LEVELS), levelA, 0)

            pltpu.make_async_copy(
                emb2d_hbm.at[pl.ds(I32(0), 8 * NUM_LEVELS), :], rows_v,
                sem).wait()

            def levelB(lv, _):
                def groupB(g, _):
                    gb = g * I32(16)
                    wx = w3_v[I32(3) * lv + I32(0), pl.ds(gb, 16)]
                    wy = w3_v[I32(3) * lv + I32(1), pl.ds(gb, 16)]
                    wz = w3_v[I32(3) * lv + I32(2), pl.ds(gb, 16)]
                    pk = [rows_v[I32(8) * lv + I32(j), pl.ds(gb, 16)]
                          for j in range(8)]
                    e0 = [plsc.bitcast(jnp.left_shift(p, I32(16)), jnp.float32)
                          for p in pk]
                    e1 = [plsc.bitcast(p & I32(-65536), jnp.float32)
                          for p in pk]
                    for f, e in ((0, e0), (1, e1)):
                        a00 = e[0] + wx * (e[4] - e[0])
                        a01 = e[1] + wx * (e[5] - e[1])
                        a10 = e[2] + wx * (e[6] - e[2])
                        a11 = e[3] + wx * (e[7] - e[3])
                        b0 = a00 + wy * (a10 - a00)
                        b1 = a01 + wy * (a11 - a01)
                        acc_v[I32(2) * lv + I32(f), pl.ds(gb, 16)] = b0 + wz * (b1 - b0)
                    return 0
                lax.fori_loop(I32(0), I32(G), groupB, 0, unroll=True)
                return 0
            lax.fori_loop(I32(0), I32(NUM_LEVELS), levelB, 0)

            pltpu.sync_copy(acc_v, enc_hbm.at[:, pl.ds(base, C)])
            return 0

        lax.fori_loop(I32(0), I32(NCHUNK), chunk_body, 0)

    return enc_kernel(pos_t, emb, emb2d, rm1f, rm1i, maskv, offv)


def _tc_mlp(enc_t, dirs_t, W1T, b1, W2T, b2, Wc1T, bc1, Wc2T, bc2, Wc3T, bc3):
    N = enc_t.shape[1]
    B = 2048
    HI = jax.lax.Precision.HIGHEST

    def body(enc_ref, dirs_ref, W1_ref, b1_ref, W2_ref, b2_ref,
             Wc1_ref, bc1_ref, Wc2_ref, bc2_ref, Wc3_ref, bc3_ref, out_ref):
        enc = enc_ref[...]
        h1 = jnp.maximum(
            jnp.dot(W1_ref[...], enc, precision=HI) + b1_ref[...], 0.0)
        h2 = jnp.dot(W2_ref[...], h1, precision=HI) + b2_ref[...]
        sigma = jnp.exp(h2[0:1, :])
        geo = h2[1:16, :]
        d = dirs_ref[...]
        x = d[0:1, :]
        y = d[1:2, :]
        z = d[2:3, :]
        norm = jnp.sqrt(x * x + y * y + z * z)
        x = x / norm
        y = y / norm
        z = z / norm
        sh = jnp.concatenate([
            jnp.full_like(x, 0.28209479177387814),
            -0.48860251190291987 * y,
            0.48860251190291987 * z,
            -0.48860251190291987 * x,
            1.0925484305920792 * x * y,
            -1.0925484305920792 * y * z,
            0.31539156525252005 * (2 * z * z - x * x - y * y),
            -1.0925484305920792 * x * z,
            0.5462742152960396 * (x * x - y * y),
            -0.5900435899266435 * y * (3 * x * x - y * y),
            2.890611442640554 * x * y * z,
            -0.4570457994644658 * y * (4 * z * z - x * x - y * y),
            0.3731763325901154 * z * (2 * z * z - 3 * x * x - 3 * y * y),
            -0.4570457994644658 * x * (4 * z * z - x * x - y * y),
            1.445305721320277 * z * (x * x - y * y),
            -0.5900435899266435 * x * (x * x - 3 * y * y),
        ], axis=0)
        c = jnp.concatenate([sh, geo], axis=0)
        c1 = jnp.maximum(jnp.dot(Wc1_ref[...], c, precision=HI) + bc1_ref[...], 0.0)
        c2 = jnp.maximum(jnp.dot(Wc2_ref[...], c1, precision=HI) + bc2_ref[...], 0.0)
        rgb = jax.nn.sigmoid(jnp.dot(Wc3_ref[...], c2, precision=HI) + bc3_ref[...])
        out_ref[...] = jnp.concatenate([rgb, sigma], axis=0)

    full = lambda shape: pl.BlockSpec(shape, lambda i: (0, 0))
    return pl.pallas_call(
        body,
        grid=(N // B,),
        in_specs=[
            pl.BlockSpec((2 * NUM_LEVELS, B), lambda i: (0, i)),
            pl.BlockSpec((3, B), lambda i: (0, i)),
            full(W1T.shape), full(b1.shape), full(W2T.shape), full(b2.shape),
            full(Wc1T.shape), full(bc1.shape), full(Wc2T.shape), full(bc2.shape),
            full(Wc3T.shape), full(bc3.shape),
        ],
        out_specs=pl.BlockSpec((4, B), lambda i: (0, i)),
        out_shape=jax.ShapeDtypeStruct((4, N), jnp.float32),
    )(enc_t, dirs_t, W1T, b1, W2T, b2, Wc1T, bc1, Wc2T, bc2, Wc3T, bc3)


def kernel(positions, directions, embeddings, W1, b1, W2, b2,
           Wc1, bc1, Wc2, bc2, Wc3, bc3):
    with jax.enable_x64(False):
        out = _kernel_x32(positions, directions, embeddings, W1, b1, W2, b2,
                          Wc1, bc1, Wc2, bc2, Wc3, bc3)
    return out.astype(jnp.float64)


def _kernel_x32(positions, directions, embeddings, W1, b1, W2, b2,
                Wc1, bc1, Wc2, bc2, Wc3, bc3):
    f32 = jnp.float32
    (positions, directions, embeddings, W1, b1, W2, b2,
     Wc1, bc1, Wc2, bc2, Wc3, bc3) = (
        a.astype(f32) for a in (positions, directions, embeddings, W1, b1,
                                W2, b2, Wc1, bc1, Wc2, bc2, Wc3, bc3))
    pos_t = positions.T
    dirs_t = directions.T
    rm1f = jnp.asarray([r - 1 for r in RES], jnp.float32)
    rm1i = jnp.asarray([r - 1 for r in RES], jnp.int32)
    maskv = jnp.asarray(MASKS, jnp.int32)
    offv = jnp.asarray(OFF, jnp.int32)
    emb_packed = jax.lax.bitcast_convert_type(
        embeddings.astype(jnp.bfloat16), jnp.int32)
    emb2d = emb_packed.reshape(-1, 128)
    enc_t = _sc_encode(pos_t, emb_packed, emb2d, rm1f, rm1i, maskv, offv)
    out_t = _tc_mlp(
        enc_t, dirs_t,
        W1.T, b1.reshape(-1, 1), W2.T, b2.reshape(-1, 1),
        Wc1.T, bc1.reshape(-1, 1), Wc2.T, bc2.reshape(-1, 1),
        Wc3.T, bc3.reshape(-1, 1),
    )
    return out_t.T


# speedup vs baseline: 50.5827x; 1.1474x over previous
import functools

import jax
import jax.numpy as jnp
import numpy as np
from jax import lax
from jax.experimental import pallas as pl
from jax.experimental.pallas import tpu as pltpu
from jax.experimental.pallas import tpu_sc as plsc

NUM_LEVELS = 16
BASE_RES = 16
FINEST = 512
LOG2_HASH = 19
FEAT = 2
HASHMAP = 2 ** LOG2_HASH
RES = []
OFF = []
_total = 0
for _lv in range(NUM_LEVELS):
    _r = min(int(BASE_RES * (2.0 ** _lv)), FINEST)
    RES.append(_r)
    OFF.append(_total)
    _total += min(_r ** 3, HASHMAP)
TOTAL_PARAMS = _total
MASKS = [min(r ** 3, HASHMAP) - 1 for r in RES]

HX = np.int32(np.uint32(73856093) & 0xFFFFFFFF)
HY = np.int32(19349663)
HZ = np.int32(np.uint32(83492791) & 0xFFFFFFFF)
I32 = np.int32

NW = 32
C = 128
K = 8 * C
G = C // 16


def _sc_encode(pos_t, emb, emb2d, rm1f, rm1i, maskv, offv):
    N = pos_t.shape[1]
    NP = N // NW
    NCHUNK = NP // C
    mesh = plsc.VectorSubcoreMesh(core_axis_name="c", subcore_axis_name="s")

    @functools.partial(
        pl.kernel,
        mesh=mesh,
        out_type=jax.ShapeDtypeStruct((2 * NUM_LEVELS, N), jnp.float32),
        scratch_types=[
            pltpu.VMEM((3, C), jnp.float32),
            pltpu.VMEM((3, C), jnp.float32),
            pltpu.VMEM((8 * NUM_LEVELS, C), jnp.int32),
            pltpu.VMEM((8 * NUM_LEVELS, C), jnp.int32),
            pltpu.VMEM((3 * NUM_LEVELS, C), jnp.float32),
            pltpu.VMEM((3 * NUM_LEVELS, C), jnp.float32),
            pltpu.VMEM((8 * NUM_LEVELS, C), jnp.int32),
            pltpu.VMEM((8 * NUM_LEVELS, C), jnp.int32),
            pltpu.VMEM((2 * NUM_LEVELS, C), jnp.float32),
            pltpu.VMEM((2 * NUM_LEVELS, C), jnp.float32),
            pltpu.VMEM((16,), jnp.float32),
            pltpu.VMEM((16,), jnp.int32),
            pltpu.VMEM((16,), jnp.int32),
            pltpu.VMEM((16,), jnp.int32),
            pltpu.SemaphoreType.DMA,
            pltpu.SemaphoreType.DMA,
        ],
        compiler_params=pltpu.CompilerParams(needs_layout_passes=False),
    )
    def enc_kernel(pos_hbm, emb_hbm, emb2d_hbm, rm1f_hbm, rm1i_hbm, mask_hbm,
                   off_hbm, enc_hbm, pos0_v, pos1_v, idx0_v, idx1_v,
                   w30_v, w31_v, rows0_v, rows1_v, acc0_v, acc1_v,
                   rm1f_v, rm1i_v, mask_v, off_v, sem0, sem1):
        wid = lax.axis_index("s") * 2 + lax.axis_index("c")
        base0 = wid * I32(NP)
        pltpu.sync_copy(rm1f_hbm, rm1f_v)
        pltpu.sync_copy(rm1i_hbm, rm1i_v)
        pltpu.sync_copy(mask_hbm, mask_v)
        pltpu.sync_copy(off_hbm, off_v)
        b0 = (pos0_v, idx0_v, w30_v, rows0_v, acc0_v, sem0)
        b1 = (pos1_v, idx1_v, w31_v, rows1_v, acc1_v, sem1)

        def phaseA(chunk, buf):
            pos_v, idx_v, w3_v, rows_v, _, sem = buf
            base = base0 + chunk * I32(C)
            pltpu.sync_copy(pos_hbm.at[:, pl.ds(base, C)], pos_v)

            def norm_body(g, _):
                gb = g * I32(16)
                for d in range(3):
                    p = pos_v[d, pl.ds(gb, 16)]
                    p01 = jnp.minimum(jnp.maximum((p + 1.0) * 0.5, 0.0), 1.0)
                    pos_v[d, pl.ds(gb, 16)] = p01
                return 0
            lax.fori_loop(I32(0), I32(G), norm_body, 0, unroll=True)

            def levelA(lv, _):
                lvv = jnp.full((16,), lv, jnp.int32)
                rm1f_b = plsc.load_gather(rm1f_v, [lvv])
                rm1i_b = plsc.load_gather(rm1i_v, [lvv])
                mask_b = plsc.load_gather(mask_v, [lvv])
                off_b = plsc.load_gather(off_v, [lvv])

                def groupA(g, _):
                    gb = g * I32(16)
                    px = pos_v[0, pl.ds(gb, 16)]
                    py = pos_v[1, pl.ds(gb, 16)]
                    pz = pos_v[2, pl.ds(gb, 16)]
                    psx = px * rm1f_b
                    psy = py * rm1f_b
                    psz = pz * rm1f_b
                    pgx = psx.astype(jnp.int32)
                    pgy = psy.astype(jnp.int32)
                    pgz = psz.astype(jnp.int32)
                    w3_v[I32(3) * lv + I32(0), pl.ds(gb, 16)] = psx - pgx.astype(jnp.float32)
                    w3_v[I32(3) * lv + I32(1), pl.ds(gb, 16)] = psy - pgy.astype(jnp.float32)
                    w3_v[I32(3) * lv + I32(2), pl.ds(gb, 16)] = psz - pgz.astype(jnp.float32)
                    x0 = pgx * HX
                    x1 = jnp.minimum(pgx + I32(1), rm1i_b) * HX
                    y0 = pgy * HY
                    y1 = jnp.minimum(pgy + I32(1), rm1i_b) * HY
                    z0 = pgz * HZ
                    z1 = jnp.minimum(pgz + I32(1), rm1i_b) * HZ
                    xy = (x0 + y0, x0 + y1, x1 + y0, x1 + y1)
                    zz = (z0, z1)
                    for j in range(8):
                        h = (xy[j >> 1] + zz[j & 1]) & mask_b
                        idx_v[I32(8) * lv + I32(j), pl.ds(gb, 16)] = h + off_b
                    return 0
                lax.fori_loop(I32(0), I32(G), groupA, 0, unroll=True)
                for j in range(8):
                    r = I32(8) * lv + I32(j)
                    pltpu.async_copy(emb_hbm.at[idx_v.at[r]], rows_v.at[r], sem)
                return 0
            lax.fori_loop(I32(0), I32(NUM_LEVELS), levelA, 0)

        def phaseB(chunk, buf):
            _, _, w3_v, rows_v, acc_v, sem = buf
            base = base0 + chunk * I32(C)
            pltpu.make_async_copy(
                emb2d_hbm.at[pl.ds(I32(0), 8 * NUM_LEVELS), :], rows_v,
                sem).wait()

            def levelB(lv, _):
                def groupB(g, _):
                    gb = g * I32(16)
                    wx = w3_v[I32(3) * lv + I32(0), pl.ds(gb, 16)]
                    wy = w3_v[I32(3) * lv + I32(1), pl.ds(gb, 16)]
                    wz = w3_v[I32(3) * lv + I32(2), pl.ds(gb, 16)]
                    pk = [rows_v[I32(8) * lv + I32(j), pl.ds(gb, 16)]
                          for j in range(8)]
                    e0 = [plsc.bitcast(jnp.left_shift(p, I32(16)), jnp.float32)
                          for p in pk]
                    e1 = [plsc.bitcast(p & I32(-65536), jnp.float32)
                          for p in pk]
                    for f, e in ((0, e0), (1, e1)):
                        a00 = e[0] + wx * (e[4] - e[0])
                        a01 = e[1] + wx * (e[5] - e[1])
                        a10 = e[2] + wx * (e[6] - e[2])
                        a11 = e[3] + wx * (e[7] - e[3])
                        b0_ = a00 + wy * (a10 - a00)
                        b1_ = a01 + wy * (a11 - a01)
                        acc_v[I32(2) * lv + I32(f), pl.ds(gb, 16)] = b0_ + wz * (b1_ - b0_)
                    return 0
                lax.fori_loop(I32(0), I32(G), groupB, 0, unroll=True)
                return 0
            lax.fori_loop(I32(0), I32(NUM_LEVELS), levelB, 0)

            pltpu.sync_copy(acc_v, enc_hbm.at[:, pl.ds(base, C)])

        phaseA(I32(0), b0)

        def pipe_body(k2, _):
            e = k2 * I32(2)
            phaseA(e + I32(1), b1)
            phaseB(e, b0)
            phaseA(e + I32(2), b0)
            phaseB(e + I32(1), b1)
            return 0
        lax.fori_loop(I32(0), I32(NCHUNK // 2 - 1), pipe_body, 0)

        phaseA(I32(NCHUNK - 1), b1)
        phaseB(I32(NCHUNK - 2), b0)
        phaseB(I32(NCHUNK - 1), b1)

    return enc_kernel(pos_t, emb, emb2d, rm1f, rm1i, maskv, offv)


def _tc_mlp(enc_t, dirs_t, W1T, b1, W2T, b2, Wc1T, bc1, Wc2T, bc2, Wc3T, bc3):
    N = enc_t.shape[1]
    B = 2048
    HI = jax.lax.Precision.HIGHEST

    def body(enc_ref, dirs_ref, W1_ref, b1_ref, W2_ref, b2_ref,
             Wc1_ref, bc1_ref, Wc2_ref, bc2_ref, Wc3_ref, bc3_ref, out_ref):
        enc = enc_ref[...]
        h1 = jnp.maximum(
            jnp.dot(W1_ref[...], enc, precision=HI) + b1_ref[...], 0.0)
        h2 = jnp.dot(W2_ref[...], h1, precision=HI) + b2_ref[...]
        sigma = jnp.exp(h2[0:1, :])
        geo = h2[1:16, :]
        d = dirs_ref[...]
        x = d[0:1, :]
        y = d[1:2, :]
        z = d[2:3, :]
        norm = jnp.sqrt(x * x + y * y + z * z)
        x = x / norm
        y = y / norm
        z = z / norm
        sh = jnp.concatenate([
            jnp.full_like(x, 0.28209479177387814),
            -0.48860251190291987 * y,
            0.48860251190291987 * z,
            -0.48860251190291987 * x,
            1.0925484305920792 * x * y,
            -1.0925484305920792 * y * z,
            0.31539156525252005 * (2 * z * z - x * x - y * y),
            -1.0925484305920792 * x * z,
            0.5462742152960396 * (x * x - y * y),
            -0.5900435899266435 * y * (3 * x * x - y * y),
            2.890611442640554 * x * y * z,
            -0.4570457994644658 * y * (4 * z * z - x * x - y * y),
            0.3731763325901154 * z * (2 * z * z - 3 * x * x - 3 * y * y),
            -0.4570457994644658 * x * (4 * z * z - x * x - y * y),
            1.445305721320277 * z * (x * x - y * y),
            -0.5900435899266435 * x * (x * x - 3 * y * y),
        ], axis=0)
        c = jnp.concatenate([sh, geo], axis=0)
        c1 = jnp.maximum(jnp.dot(Wc1_ref[...], c, precision=HI) + bc1_ref[...], 0.0)
        c2 = jnp.maximum(jnp.dot(Wc2_ref[...], c1, precision=HI) + bc2_ref[...], 0.0)
        rgb = jax.nn.sigmoid(jnp.dot(Wc3_ref[...], c2, precision=HI) + bc3_ref[...])
        out_ref[...] = jnp.concatenate([rgb, sigma], axis=0)

    full = lambda shape: pl.BlockSpec(shape, lambda i: (0, 0))
    return pl.pallas_call(
        body,
        grid=(N // B,),
        in_specs=[
            pl.BlockSpec((2 * NUM_LEVELS, B), lambda i: (0, i)),
            pl.BlockSpec((3, B), lambda i: (0, i)),
            full(W1T.shape), full(b1.shape), full(W2T.shape), full(b2.shape),
            full(Wc1T.shape), full(bc1.shape), full(Wc2T.shape), full(bc2.shape),
            full(Wc3T.shape), full(bc3.shape),
        ],
        out_specs=pl.BlockSpec((4, B), lambda i: (0, i)),
        out_shape=jax.ShapeDtypeStruct((4, N), jnp.float32),
    )(enc_t, dirs_t, W1T, b1, W2T, b2, Wc1T, bc1, Wc2T, bc2, Wc3T, bc3)


def kernel(positions, directions, embeddings, W1, b1, W2, b2,
           Wc1, bc1, Wc2, bc2, Wc3, bc3):
    with jax.enable_x64(False):
        out = _kernel_x32(positions, directions, embeddings, W1, b1, W2, b2,
                          Wc1, bc1, Wc2, bc2, Wc3, bc3)
    return out.astype(jnp.float64)


def _kernel_x32(positions, directions, embeddings, W1, b1, W2, b2,
                Wc1, bc1, Wc2, bc2, Wc3, bc3):
    f32 = jnp.float32
    (positions, directions, embeddings, W1, b1, W2, b2,
     Wc1, bc1, Wc2, bc2, Wc3, bc3) = (
        a.astype(f32) for a in (positions, directions, embeddings, W1, b1,
                                W2, b2, Wc1, bc1, Wc2, bc2, Wc3, bc3))
    pos_t = positions.T
    dirs_t = directions.T
    rm1f = jnp.asarray([r - 1 for r in RES], jnp.float32)
    rm1i = jnp.asarray([r - 1 for r in RES], jnp.int32)
    maskv = jnp.asarray(MASKS, jnp.int32)
    offv = jnp.asarray(OFF, jnp.int32)
    emb_packed = jax.lax.bitcast_convert_type(
        embeddings.astype(jnp.bfloat16), jnp.int32)
    emb2d = emb_packed.reshape(-1, 128)
    enc_t = _sc_encode(pos_t, emb_packed, emb2d, rm1f, rm1i, maskv, offv)
    out_t = _tc_mlp(
        enc_t, dirs_t,
        W1.T, b1.reshape(-1, 1), W2.T, b2.reshape(-1, 1),
        Wc1.T, bc1.reshape(-1, 1), Wc2.T, bc2.reshape(-1, 1),
        Wc3.T, bc3.reshape(-1, 1),
    )
    return out_t.T


# speedup vs baseline: 55.9128x; 1.1054x over previous
import functools

import jax
import jax.numpy as jnp
import numpy as np
from jax import lax
from jax.experimental import pallas as pl
from jax.experimental.pallas import tpu as pltpu
from jax.experimental.pallas import tpu_sc as plsc

NUM_LEVELS = 16
BASE_RES = 16
FINEST = 512
LOG2_HASH = 19
FEAT = 2
HASHMAP = 2 ** LOG2_HASH
RES = []
OFF = []
_total = 0
for _lv in range(NUM_LEVELS):
    _r = min(int(BASE_RES * (2.0 ** _lv)), FINEST)
    RES.append(_r)
    OFF.append(_total)
    _total += min(_r ** 3, HASHMAP)
TOTAL_PARAMS = _total
MASKS = [min(r ** 3, HASHMAP) - 1 for r in RES]

HX = np.int32(np.uint32(73856093) & 0xFFFFFFFF)
HY = np.int32(19349663)
HZ = np.int32(np.uint32(83492791) & 0xFFFFFFFF)
I32 = np.int32

NW = 32
C = 128
K = 8 * C
G = C // 16


def _sc_encode(pos_t, emb, emb2d, rm1f, rm1i, maskv, offv):
    N = pos_t.shape[1]
    NP = N // NW
    NCHUNK = NP // C
    mesh = plsc.VectorSubcoreMesh(core_axis_name="c", subcore_axis_name="s")

    @functools.partial(
        pl.kernel,
        mesh=mesh,
        out_type=jax.ShapeDtypeStruct((2 * NUM_LEVELS, N), jnp.float32),
        scratch_types=[
            pltpu.VMEM((3, C), jnp.float32),
            pltpu.VMEM((3, C), jnp.float32),
            pltpu.VMEM((8 * NUM_LEVELS, C), jnp.int32),
            pltpu.VMEM((8 * NUM_LEVELS, C), jnp.int32),
            pltpu.VMEM((3 * NUM_LEVELS, C), jnp.float32),
            pltpu.VMEM((3 * NUM_LEVELS, C), jnp.float32),
            pltpu.VMEM((8 * NUM_LEVELS, C), jnp.int32),
            pltpu.VMEM((8 * NUM_LEVELS, C), jnp.int32),
            pltpu.VMEM((2 * NUM_LEVELS, C), jnp.float32),
            pltpu.VMEM((2 * NUM_LEVELS, C), jnp.float32),
            pltpu.VMEM((16,), jnp.float32),
            pltpu.VMEM((16,), jnp.int32),
            pltpu.VMEM((16,), jnp.int32),
            pltpu.VMEM((16,), jnp.int32),
            pltpu.SemaphoreType.DMA,
            pltpu.SemaphoreType.DMA,
        ],
        compiler_params=pltpu.CompilerParams(needs_layout_passes=False),
    )
    def enc_kernel(pos_hbm, emb_hbm, emb2d_hbm, rm1f_hbm, rm1i_hbm, mask_hbm,
                   off_hbm, enc_hbm, pos0_v, pos1_v, idx0_v, idx1_v,
                   w30_v, w31_v, rows0_v, rows1_v, acc0_v, acc1_v,
                   rm1f_v, rm1i_v, mask_v, off_v, sem0, sem1):
        wid = lax.axis_index("s") * 2 + lax.axis_index("c")
        base0 = wid * I32(NP)
        pltpu.sync_copy(rm1f_hbm, rm1f_v)
        pltpu.sync_copy(rm1i_hbm, rm1i_v)
        pltpu.sync_copy(mask_hbm, mask_v)
        pltpu.sync_copy(off_hbm, off_v)
        b0 = (pos0_v, idx0_v, w30_v, rows0_v, acc0_v, sem0)
        b1 = (pos1_v, idx1_v, w31_v, rows1_v, acc1_v, sem1)

        def phaseA(chunk, buf):
            pos_v, idx_v, w3_v, rows_v, _, sem = buf
            base = base0 + chunk * I32(C)
            pltpu.sync_copy(pos_hbm.at[:, pl.ds(base, C)], pos_v)

            def norm_body(g, _):
                gb = g * I32(16)
                for d in range(3):
                    p = pos_v[d, pl.ds(gb, 16)]
                    p01 = jnp.minimum(jnp.maximum((p + 1.0) * 0.5, 0.0), 1.0)
                    pos_v[d, pl.ds(gb, 16)] = p01
                return 0
            lax.fori_loop(I32(0), I32(G), norm_body, 0, unroll=True)

            def levelA(lv, _):
                lvv = jnp.full((16,), lv, jnp.int32)
                rm1f_b = plsc.load_gather(rm1f_v, [lvv])
                rm1i_b = plsc.load_gather(rm1i_v, [lvv])
                mask_b = plsc.load_gather(mask_v, [lvv])
                off_b = plsc.load_gather(off_v, [lvv])

                def groupA(g, _):
                    gb = g * I32(16)
                    px = pos_v[0, pl.ds(gb, 16)]
                    py = pos_v[1, pl.ds(gb, 16)]
                    pz = pos_v[2, pl.ds(gb, 16)]
                    psx = px * rm1f_b
                    psy = py * rm1f_b
                    psz = pz * rm1f_b
                    pgx = psx.astype(jnp.int32)
                    pgy = psy.astype(jnp.int32)
                    pgz = psz.astype(jnp.int32)
                    w3_v[I32(3) * lv + I32(0), pl.ds(gb, 16)] = psx - pgx.astype(jnp.float32)
                    w3_v[I32(3) * lv + I32(1), pl.ds(gb, 16)] = psy - pgy.astype(jnp.float32)
                    w3_v[I32(3) * lv + I32(2), pl.ds(gb, 16)] = psz - pgz.astype(jnp.float32)
                    x0 = pgx * HX
                    x1 = jnp.minimum(pgx + I32(1), rm1i_b) * HX
                    y0 = pgy * HY
                    y1 = jnp.minimum(pgy + I32(1), rm1i_b) * HY
                    z0 = pgz * HZ
                    z1 = jnp.minimum(pgz + I32(1), rm1i_b) * HZ
                    xy = (x0 + y0, x0 + y1, x1 + y0, x1 + y1)
                    zz = (z0, z1)
                    for j in range(8):
                        h = (xy[j >> 1] + zz[j & 1]) & mask_b
                        idx_v[I32(8) * lv + I32(j), pl.ds(gb, 16)] = h + off_b
                    return 0
                lax.fori_loop(I32(0), I32(G), groupA, 0, unroll=True)
                for j in range(8):
                    r = I32(8) * lv + I32(j)
                    pltpu.async_copy(emb_hbm.at[idx_v.at[r]], rows_v.at[r], sem)
                return 0
            lax.fori_loop(I32(0), I32(NUM_LEVELS), levelA, 0)

        def phaseB(chunk, buf):
            _, _, w3_v, rows_v, acc_v, sem = buf
            base = base0 + chunk * I32(C)
            pltpu.make_async_copy(
                emb2d_hbm.at[pl.ds(I32(0), 8 * NUM_LEVELS), :], rows_v,
                sem).wait()

            def levelB(lv, _):
                def groupB(g, _):
                    gb = g * I32(16)
                    wx = w3_v[I32(3) * lv + I32(0), pl.ds(gb, 16)]
                    wy = w3_v[I32(3) * lv + I32(1), pl.ds(gb, 16)]
                    wz = w3_v[I32(3) * lv + I32(2), pl.ds(gb, 16)]
                    pk = [rows_v[I32(8) * lv + I32(j), pl.ds(gb, 16)]
                          for j in range(8)]
                    e0 = [plsc.bitcast(jnp.left_shift(p, I32(16)), jnp.float32)
                          for p in pk]
                    e1 = [plsc.bitcast(p & I32(-65536), jnp.float32)
                          for p in pk]
                    for f, e in ((0, e0), (1, e1)):
                        a00 = e[0] + wx * (e[4] - e[0])
                        a01 = e[1] + wx * (e[5] - e[1])
                        a10 = e[2] + wx * (e[6] - e[2])
                        a11 = e[3] + wx * (e[7] - e[3])
                        b0_ = a00 + wy * (a10 - a00)
                        b1_ = a01 + wy * (a11 - a01)
                        acc_v[I32(2) * lv + I32(f), pl.ds(gb, 16)] = b0_ + wz * (b1_ - b0_)
                    return 0
                lax.fori_loop(I32(0), I32(G), groupB, 0, unroll=True)
                return 0
            lax.fori_loop(I32(0), I32(NUM_LEVELS), levelB, 0)

            pltpu.sync_copy(acc_v, enc_hbm.at[:, pl.ds(base, C)])

        phaseA(I32(0), b0)

        def pipe_body(k2, _):
            e = k2 * I32(2)
            phaseA(e + I32(1), b1)
            phaseB(e, b0)
            phaseA(e + I32(2), b0)
            phaseB(e + I32(1), b1)
            return 0
        lax.fori_loop(I32(0), I32(NCHUNK // 2 - 1), pipe_body, 0)

        phaseA(I32(NCHUNK - 1), b1)
        phaseB(I32(NCHUNK - 2), b0)
        phaseB(I32(NCHUNK - 1), b1)

    return enc_kernel(pos_t, emb, emb2d, rm1f, rm1i, maskv, offv)


def _tc_mlp(enc_t, dirs_t, W1T, b1, W2T, b2, Wc1T, bc1, Wc2T, bc2, Wc3T, bc3):
    N = enc_t.shape[1]
    B = 4096

    def body(enc_ref, dirs_ref, W1_ref, b1_ref, W2_ref, b2_ref,
             Wc1_ref, bc1_ref, Wc2_ref, bc2_ref, Wc3_ref, bc3_ref, out_ref,
             c_ref):
        enc = enc_ref[...]
        h1 = jnp.maximum(jnp.dot(W1_ref[...], enc) + b1_ref[...], 0.0)
        h2 = jnp.dot(W2_ref[...], h1) + b2_ref[...]
        d = dirs_ref[...]
        x = d[0:1, :]
        y = d[1:2, :]
        z = d[2:3, :]
        norm = jnp.sqrt(x * x + y * y + z * z)
        x = x / norm
        y = y / norm
        z = z / norm
        xx = x * x
        yy = y * y
        zz = z * z
        c_ref[0:1, :] = jnp.full_like(x, 0.28209479177387814)
        c_ref[1:2, :] = -0.48860251190291987 * y
        c_ref[2:3, :] = 0.48860251190291987 * z
        c_ref[3:4, :] = -0.48860251190291987 * x
        c_ref[4:5, :] = 1.0925484305920792 * x * y
        c_ref[5:6, :] = -1.0925484305920792 * y * z
        c_ref[6:7, :] = 0.31539156525252005 * (2 * zz - xx - yy)
        c_ref[7:8, :] = -1.0925484305920792 * x * z
        c_ref[8:9, :] = 0.5462742152960396 * (xx - yy)
        c_ref[9:10, :] = -0.5900435899266435 * y * (3 * xx - yy)
        c_ref[10:11, :] = 2.890611442640554 * x * y * z
        c_ref[11:12, :] = -0.4570457994644658 * y * (4 * zz - xx - yy)
        c_ref[12:13, :] = 0.3731763325901154 * z * (2 * zz - 3 * xx - 3 * yy)
        c_ref[13:14, :] = -0.4570457994644658 * x * (4 * zz - xx - yy)
        c_ref[14:15, :] = 1.445305721320277 * z * (xx - yy)
        c_ref[15:16, :] = -0.5900435899266435 * x * (xx - 3 * yy)
        c_ref[16:31, :] = h2[1:16, :]
        c_ref[31:32, :] = jnp.zeros_like(x)
        c = c_ref[...]
        c1 = jnp.maximum(jnp.dot(Wc1_ref[...], c) + bc1_ref[...], 0.0)
        c2 = jnp.maximum(jnp.dot(Wc2_ref[...], c1) + bc2_ref[...], 0.0)
        out_ref[0:3, :] = jax.nn.sigmoid(
            jnp.dot(Wc3_ref[...], c2) + bc3_ref[...])
        out_ref[3:4, :] = jnp.exp(h2[0:1, :])

    full = lambda shape: pl.BlockSpec(shape, lambda i: (0, 0))
    return pl.pallas_call(
        body,
        grid=(N // B,),
        in_specs=[
            pl.BlockSpec((2 * NUM_LEVELS, B), lambda i: (0, i)),
            pl.BlockSpec((3, B), lambda i: (0, i)),
            full(W1T.shape), full(b1.shape), full(W2T.shape), full(b2.shape),
            full(Wc1T.shape), full(bc1.shape), full(Wc2T.shape), full(bc2.shape),
            full(Wc3T.shape), full(bc3.shape),
        ],
        out_specs=pl.BlockSpec((4, B), lambda i: (0, i)),
        out_shape=jax.ShapeDtypeStruct((4, N), jnp.float32),
        scratch_shapes=[pltpu.VMEM((32, B), jnp.float32)],
    )(enc_t, dirs_t, W1T, b1, W2T, b2, Wc1T, bc1, Wc2T, bc2, Wc3T, bc3)


def kernel(positions, directions, embeddings, W1, b1, W2, b2,
           Wc1, bc1, Wc2, bc2, Wc3, bc3):
    with jax.enable_x64(False):
        out = _kernel_x32(positions, directions, embeddings, W1, b1, W2, b2,
                          Wc1, bc1, Wc2, bc2, Wc3, bc3)
    return out.astype(jnp.float64)


def _kernel_x32(positions, directions, embeddings, W1, b1, W2, b2,
                Wc1, bc1, Wc2, bc2, Wc3, bc3):
    f32 = jnp.float32
    (positions, directions, embeddings, W1, b1, W2, b2,
     Wc1, bc1, Wc2, bc2, Wc3, bc3) = (
        a.astype(f32) for a in (positions, directions, embeddings, W1, b1,
                                W2, b2, Wc1, bc1, Wc2, bc2, Wc3, bc3))
    pos_t = positions.T
    dirs_t = directions.T
    rm1f = jnp.asarray([r - 1 for r in RES], jnp.float32)
    rm1i = jnp.asarray([r - 1 for r in RES], jnp.int32)
    maskv = jnp.asarray(MASKS, jnp.int32)
    offv = jnp.asarray(OFF, jnp.int32)
    emb_packed = jax.lax.bitcast_convert_type(
        embeddings.astype(jnp.bfloat16), jnp.int32)
    emb2d = emb_packed.reshape(-1, 128)
    enc_t = _sc_encode(pos_t, emb_packed, emb2d, rm1f, rm1i, maskv, offv)
    out_t = _tc_mlp(
        enc_t, dirs_t,
        W1.T, b1.reshape(-1, 1), W2.T, b2.reshape(-1, 1),
        jnp.pad(Wc1.T, ((0, 0), (0, 1))), bc1.reshape(-1, 1),
        Wc2.T, bc2.reshape(-1, 1),
        Wc3.T, bc3.reshape(-1, 1),
    )
    return out_t.T


# speedup vs baseline: 65.0978x; 1.1643x over previous
import functools

import jax
import jax.numpy as jnp
import numpy as np
from jax import lax
from jax.experimental import pallas as pl
from jax.experimental.pallas import tpu as pltpu
from jax.experimental.pallas import tpu_sc as plsc

NUM_LEVELS = 16
BASE_RES = 16
FINEST = 512
LOG2_HASH = 19
FEAT = 2
HASHMAP = 2 ** LOG2_HASH
RES = []
OFF = []
_total = 0
for _lv in range(NUM_LEVELS):
    _r = min(int(BASE_RES * (2.0 ** _lv)), FINEST)
    RES.append(_r)
    OFF.append(_total)
    _total += min(_r ** 3, HASHMAP)
TOTAL_PARAMS = _total
MASKS = [min(r ** 3, HASHMAP) - 1 for r in RES]

HX = np.int32(np.uint32(73856093) & 0xFFFFFFFF)
HY = np.int32(19349663)
HZ = np.int32(np.uint32(83492791) & 0xFFFFFFFF)
I32 = np.int32

NW = 32
C = 128
K = 8 * C
G = C // 16


def _sc_encode(pos_t, emb, emb2d, rm1f, rm1i, maskv, offv):
    N = pos_t.shape[1]
    NP = N // NW
    NCHUNK = NP // C
    mesh = plsc.VectorSubcoreMesh(core_axis_name="c", subcore_axis_name="s")

    @functools.partial(
        pl.kernel,
        mesh=mesh,
        out_type=jax.ShapeDtypeStruct((2 * NUM_LEVELS, N), jnp.float32),
        scratch_types=[
            pltpu.VMEM((3, C), jnp.float32),
            pltpu.VMEM((3, C), jnp.float32),
            pltpu.VMEM((8 * NUM_LEVELS, C), jnp.int32),
            pltpu.VMEM((8 * NUM_LEVELS, C), jnp.int32),
            pltpu.VMEM((3 * NUM_LEVELS, C), jnp.float32),
            pltpu.VMEM((3 * NUM_LEVELS, C), jnp.float32),
            pltpu.VMEM((8 * NUM_LEVELS, C), jnp.int32),
            pltpu.VMEM((8 * NUM_LEVELS, C), jnp.int32),
            pltpu.VMEM((2 * NUM_LEVELS, C), jnp.float32),
            pltpu.VMEM((2 * NUM_LEVELS, C), jnp.float32),
            pltpu.VMEM((36864,), jnp.int32),
            pltpu.VMEM((16,), jnp.float32),
            pltpu.VMEM((16,), jnp.int32),
            pltpu.VMEM((16,), jnp.int32),
            pltpu.VMEM((16,), jnp.int32),
            pltpu.SemaphoreType.DMA,
            pltpu.SemaphoreType.DMA,
        ],
        compiler_params=pltpu.CompilerParams(needs_layout_passes=False),
    )
    def enc_kernel(pos_hbm, emb_hbm, emb2d_hbm, rm1f_hbm, rm1i_hbm, mask_hbm,
                   off_hbm, enc_hbm, pos0_v, pos1_v, idx0_v, idx1_v,
                   w30_v, w31_v, rows0_v, rows1_v, acc0_v, acc1_v,
                   t01_v, rm1f_v, rm1i_v, mask_v, off_v, sem0, sem1):
        wid = lax.axis_index("s") * 2 + lax.axis_index("c")
        base0 = wid * I32(NP)
        pltpu.sync_copy(rm1f_hbm, rm1f_v)
        pltpu.sync_copy(rm1i_hbm, rm1i_v)
        pltpu.sync_copy(mask_hbm, mask_v)
        pltpu.sync_copy(off_hbm, off_v)
        pltpu.sync_copy(emb_hbm.at[pl.ds(I32(0), 36864)], t01_v)
        b0 = (pos0_v, idx0_v, w30_v, rows0_v, acc0_v, sem0)
        b1 = (pos1_v, idx1_v, w31_v, rows1_v, acc1_v, sem1)

        def phaseA(chunk, buf):
            pos_v, idx_v, w3_v, rows_v, _, sem = buf
            base = base0 + chunk * I32(C)
            pltpu.sync_copy(pos_hbm.at[:, pl.ds(base, C)], pos_v)

            def norm_body(g, _):
                gb = g * I32(16)
                for d in range(3):
                    p = pos_v[d, pl.ds(gb, 16)]
                    p01 = jnp.minimum(jnp.maximum((p + 1.0) * 0.5, 0.0), 1.0)
                    pos_v[d, pl.ds(gb, 16)] = p01
                return 0
            lax.fori_loop(I32(0), I32(G), norm_body, 0, unroll=True)

            def levelA(lv, _):
                lvv = jnp.full((16,), lv, jnp.int32)
                rbase8 = I32(8) * lv - I32(16)
                rm1f_b = plsc.load_gather(rm1f_v, [lvv])
                rm1i_b = plsc.load_gather(rm1i_v, [lvv])
                mask_b = plsc.load_gather(mask_v, [lvv])
                off_b = plsc.load_gather(off_v, [lvv])

                def groupA(g, _):
                    gb = g * I32(16)
                    px = pos_v[0, pl.ds(gb, 16)]
                    py = pos_v[1, pl.ds(gb, 16)]
                    pz = pos_v[2, pl.ds(gb, 16)]
                    psx = px * rm1f_b
                    psy = py * rm1f_b
                    psz = pz * rm1f_b
                    pgx = psx.astype(jnp.int32)
                    pgy = psy.astype(jnp.int32)
                    pgz = psz.astype(jnp.int32)
                    w3_v[I32(3) * lv + I32(0), pl.ds(gb, 16)] = psx - pgx.astype(jnp.float32)
                    w3_v[I32(3) * lv + I32(1), pl.ds(gb, 16)] = psy - pgy.astype(jnp.float32)
                    w3_v[I32(3) * lv + I32(2), pl.ds(gb, 16)] = psz - pgz.astype(jnp.float32)
                    x0 = pgx * HX
                    x1 = jnp.minimum(pgx + I32(1), rm1i_b) * HX
                    y0 = pgy * HY
                    y1 = jnp.minimum(pgy + I32(1), rm1i_b) * HY
                    z0 = pgz * HZ
                    z1 = jnp.minimum(pgz + I32(1), rm1i_b) * HZ
                    xy = (x0 + y0, x0 + y1, x1 + y0, x1 + y1)
                    zz = (z0, z1)
                    for j in range(8):
                        h = (xy[j >> 1] + zz[j & 1]) & mask_b
                        idx_v[rbase8 + I32(j), pl.ds(gb, 16)] = h + off_b
                    return 0
                lax.fori_loop(I32(0), I32(G), groupA, 0, unroll=True)
                for j in range(8):
                    r = rbase8 + I32(j)
                    pltpu.async_copy(emb_hbm.at[idx_v.at[r]], rows_v.at[r], sem)
                return 0
            lax.fori_loop(I32(2), I32(NUM_LEVELS), levelA, 0)

            for lv01 in (0, 1):
                rm1f_c = np.float32(RES[lv01] - 1)
                rm1i_c = I32(RES[lv01] - 1)
                mask_c = I32(MASKS[lv01])
                off_c = I32(OFF[lv01])
                acc_v = buf[4]

                def group01(g, _, lv01=lv01, rm1f_c=rm1f_c, rm1i_c=rm1i_c,
                            mask_c=mask_c, off_c=off_c, acc_v=acc_v,
                            pos_v=pos_v):
                    gb = g * I32(16)
                    px = pos_v[0, pl.ds(gb, 16)]
                    py = pos_v[1, pl.ds(gb, 16)]
                    pz = pos_v[2, pl.ds(gb, 16)]
                    psx = px * rm1f_c
                    psy = py * rm1f_c
                    psz = pz * rm1f_c
                    pgx = psx.astype(jnp.int32)
                    pgy = psy.astype(jnp.int32)
                    pgz = psz.astype(jnp.int32)
                    wx = psx - pgx.astype(jnp.float32)
                    wy = psy - pgy.astype(jnp.float32)
                    wz = psz - pgz.astype(jnp.float32)
                    x0 = pgx * HX
                    x1 = jnp.minimum(pgx + I32(1), rm1i_c) * HX
                    y0 = pgy * HY
                    y1 = jnp.minimum(pgy + I32(1), rm1i_c) * HY
                    z0 = pgz * HZ
                    z1 = jnp.minimum(pgz + I32(1), rm1i_c) * HZ
                    xy = (x0 + y0, x0 + y1, x1 + y0, x1 + y1)
                    zs = (z0, z1)
                    pk = [plsc.load_gather(
                              t01_v,
                              [((xy[j >> 1] + zs[j & 1]) & mask_c) + off_c])
                          for j in range(8)]
                    e0 = [plsc.bitcast(jnp.left_shift(p, I32(16)), jnp.float32)
                          for p in pk]
                    e1 = [plsc.bitcast(p & I32(-65536), jnp.float32)
                          for p in pk]
                    for f, e in ((0, e0), (1, e1)):
                        a00 = e[0] + wx * (e[4] - e[0])
                        a01 = e[1] + wx * (e[5] - e[1])
                        a10 = e[2] + wx * (e[6] - e[2])
                        a11 = e[3] + wx * (e[7] - e[3])
                        b0_ = a00 + wy * (a10 - a00)
                        b1_ = a01 + wy * (a11 - a01)
                        acc_v[I32(2 * lv01 + f), pl.ds(gb, 16)] = \
                            b0_ + wz * (b1_ - b0_)
                    return 0
                lax.fori_loop(I32(0), I32(G), group01, 0, unroll=True)

        def phaseB(chunk, buf):
            _, _, w3_v, rows_v, acc_v, sem = buf
            base = base0 + chunk * I32(C)
            pltpu.make_async_copy(
                emb2d_hbm.at[pl.ds(I32(0), 8 * (NUM_LEVELS - 2)), :],
                rows_v.at[pl.ds(I32(0), 8 * (NUM_LEVELS - 2))], sem).wait()

            def levelB(lv, _):
                rbase8 = I32(8) * lv - I32(16)

                def groupB(g, _):
                    gb = g * I32(16)
                    wx = w3_v[I32(3) * lv + I32(0), pl.ds(gb, 16)]
                    wy = w3_v[I32(3) * lv + I32(1), pl.ds(gb, 16)]
                    wz = w3_v[I32(3) * lv + I32(2), pl.ds(gb, 16)]
                    pk = [rows_v[rbase8 + I32(j), pl.ds(gb, 16)]
                          for j in range(8)]
                    e0 = [plsc.bitcast(jnp.left_shift(p, I32(16)), jnp.float32)
                          for p in pk]
                    e1 = [plsc.bitcast(p & I32(-65536), jnp.float32)
                          for p in pk]
                    for f, e in ((0, e0), (1, e1)):
                        a00 = e[0] + wx * (e[4] - e[0])
                        a01 = e[1] + wx * (e[5] - e[1])
                        a10 = e[2] + wx * (e[6] - e[2])
                        a11 = e[3] + wx * (e[7] - e[3])
                        b0_ = a00 + wy * (a10 - a00)
                        b1_ = a01 + wy * (a11 - a01)
                        acc_v[I32(2) * lv + I32(f), pl.ds(gb, 16)] = b0_ + wz * (b1_ - b0_)
                    return 0
                lax.fori_loop(I32(0), I32(G), groupB, 0, unroll=True)
                return 0
            lax.fori_loop(I32(2), I32(NUM_LEVELS), levelB, 0)

            pltpu.sync_copy(acc_v, enc_hbm.at[:, pl.ds(base, C)])

        phaseA(I32(0), b0)

        def pipe_body(k2, _):
            e = k2 * I32(2)
            phaseA(e + I32(1), b1)
            phaseB(e, b0)
            phaseA(e + I32(2), b0)
            phaseB(e + I32(1), b1)
            return 0
        lax.fori_loop(I32(0), I32(NCHUNK // 2 - 1), pipe_body, 0)

        phaseA(I32(NCHUNK - 1), b1)
        phaseB(I32(NCHUNK - 2), b0)
        phaseB(I32(NCHUNK - 1), b1)

    return enc_kernel(pos_t, emb, emb2d, rm1f, rm1i, maskv, offv)


def _tc_mlp(enc_t, dirs_t, W1T, b1, W2T, b2, Wc1T, bc1, Wc2T, bc2, Wc3T, bc3):
    N = enc_t.shape[1]
    B = 4096

    def body(enc_ref, dirs_ref, W1_ref, b1_ref, W2_ref, b2_ref,
             Wc1_ref, bc1_ref, Wc2_ref, bc2_ref, Wc3_ref, bc3_ref, out_ref,
             c_ref):
        enc = enc_ref[...]
        h1 = jnp.maximum(jnp.dot(W1_ref[...], enc) + b1_ref[...], 0.0)
        h2 = jnp.dot(W2_ref[...], h1) + b2_ref[...]
        d = dirs_ref[...]
        x = d[0:1, :]
        y = d[1:2, :]
        z = d[2:3, :]
        norm = jnp.sqrt(x * x + y * y + z * z)
        x = x / norm
        y = y / norm
        z = z / norm
        xx = x * x
        yy = y * y
        zz = z * z
        c_ref[0:1, :] = jnp.full_like(x, 0.28209479177387814)
        c_ref[1:2, :] = -0.48860251190291987 * y
        c_ref[2:3, :] = 0.48860251190291987 * z
        c_ref[3:4, :] = -0.48860251190291987 * x
        c_ref[4:5, :] = 1.0925484305920792 * x * y
        c_ref[5:6, :] = -1.0925484305920792 * y * z
        c_ref[6:7, :] = 0.31539156525252005 * (2 * zz - xx - yy)
        c_ref[7:8, :] = -1.0925484305920792 * x * z
        c_ref[8:9, :] = 0.5462742152960396 * (xx - yy)
        c_ref[9:10, :] = -0.5900435899266435 * y * (3 * xx - yy)
        c_ref[10:11, :] = 2.890611442640554 * x * y * z
        c_ref[11:12, :] = -0.4570457994644658 * y * (4 * zz - xx - yy)
        c_ref[12:13, :] = 0.3731763325901154 * z * (2 * zz - 3 * xx - 3 * yy)
        c_ref[13:14, :] = -0.4570457994644658 * x * (4 * zz - xx - yy)
        c_ref[14:15, :] = 1.445305721320277 * z * (xx - yy)
        c_ref[15:16, :] = -0.5900435899266435 * x * (xx - 3 * yy)
        c_ref[16:31, :] = h2[1:16, :]
        c_ref[31:32, :] = jnp.zeros_like(x)
        c = c_ref[...]
        c1 = jnp.maximum(jnp.dot(Wc1_ref[...], c) + bc1_ref[...], 0.0)
        c2 = jnp.maximum(jnp.dot(Wc2_ref[...], c1) + bc2_ref[...], 0.0)
        out_ref[0:3, :] = jax.nn.sigmoid(
            jnp.dot(Wc3_ref[...], c2) + bc3_ref[...])
        out_ref[3:4, :] = jnp.exp(h2[0:1, :])

    full = lambda shape: pl.BlockSpec(shape, lambda i: (0, 0))
    return pl.pallas_call(
        body,
        grid=(N // B,),
        in_specs=[
            pl.BlockSpec((2 * NUM_LEVELS, B), lambda i: (0, i)),
            pl.BlockSpec((3, B), lambda i: (0, i)),
            full(W1T.shape), full(b1.shape), full(W2T.shape), full(b2.shape),
            full(Wc1T.shape), full(bc1.shape), full(Wc2T.shape), full(bc2.shape),
            full(Wc3T.shape), full(bc3.shape),
        ],
        out_specs=pl.BlockSpec((4, B), lambda i: (0, i)),
        out_shape=jax.ShapeDtypeStruct((4, N), jnp.float32),
        scratch_shapes=[pltpu.VMEM((32, B), jnp.float32)],
    )(enc_t, dirs_t, W1T, b1, W2T, b2, Wc1T, bc1, Wc2T, bc2, Wc3T, bc3)


def kernel(positions, directions, embeddings, W1, b1, W2, b2,
           Wc1, bc1, Wc2, bc2, Wc3, bc3):
    with jax.enable_x64(False):
        out = _kernel_x32(positions, directions, embeddings, W1, b1, W2, b2,
                          Wc1, bc1, Wc2, bc2, Wc3, bc3)
    return out.astype(jnp.float64)


def _kernel_x32(positions, directions, embeddings, W1, b1, W2, b2,
                Wc1, bc1, Wc2, bc2, Wc3, bc3):
    f32 = jnp.float32
    (positions, directions, embeddings, W1, b1, W2, b2,
     Wc1, bc1, Wc2, bc2, Wc3, bc3) = (
        a.astype(f32) for a in (positions, directions, embeddings, W1, b1,
                                W2, b2, Wc1, bc1, Wc2, bc2, Wc3, bc3))
    pos_t = positions.T
    dirs_t = directions.T
    rm1f = jnp.asarray([r - 1 for r in RES], jnp.float32)
    rm1i = jnp.asarray([r - 1 for r in RES], jnp.int32)
    maskv = jnp.asarray(MASKS, jnp.int32)
    offv = jnp.asarray(OFF, jnp.int32)
    emb_packed = jax.lax.bitcast_convert_type(
        embeddings.astype(jnp.bfloat16), jnp.int32)
    emb2d = emb_packed.reshape(-1, 128)
    enc_t = _sc_encode(pos_t, emb_packed, emb2d, rm1f, rm1i, maskv, offv)
    out_t = _tc_mlp(
        enc_t, dirs_t,
        W1.T, b1.reshape(-1, 1), W2.T, b2.reshape(-1, 1),
        jnp.pad(Wc1.T, ((0, 0), (0, 1))), bc1.reshape(-1, 1),
        Wc2.T, bc2.reshape(-1, 1),
        Wc3.T, bc3.reshape(-1, 1),
    )
    return out_t.T


# speedup vs baseline: 65.5471x; 1.0069x over previous
import functools

import jax
import jax.numpy as jnp
import numpy as np
from jax import lax
from jax.experimental import pallas as pl
from jax.experimental.pallas import tpu as pltpu
from jax.experimental.pallas import tpu_sc as plsc

NUM_LEVELS = 16
BASE_RES = 16
FINEST = 512
LOG2_HASH = 19
FEAT = 2
HASHMAP = 2 ** LOG2_HASH
RES = []
OFF = []
_total = 0
for _lv in range(NUM_LEVELS):
    _r = min(int(BASE_RES * (2.0 ** _lv)), FINEST)
    RES.append(_r)
    OFF.append(_total)
    _total += min(_r ** 3, HASHMAP)
TOTAL_PARAMS = _total
MASKS = [min(r ** 3, HASHMAP) - 1 for r in RES]

HX = np.int32(np.uint32(73856093) & 0xFFFFFFFF)
HY = np.int32(19349663)
HZ = np.int32(np.uint32(83492791) & 0xFFFFFFFF)
I32 = np.int32

NW = 32
C = 128
K = 8 * C
G = C // 16


def _sc_encode(pos_t, emb, emb2d, rm1f, rm1i, maskv, offv):
    N = pos_t.shape[1]
    NP = N // NW
    NCHUNK = NP // C
    mesh = plsc.VectorSubcoreMesh(core_axis_name="c", subcore_axis_name="s")

    @functools.partial(
        pl.kernel,
        mesh=mesh,
        out_type=jax.ShapeDtypeStruct((2 * NUM_LEVELS, N), jnp.float32),
        scratch_types=[
            pltpu.VMEM((3, C), jnp.float32),
            pltpu.VMEM((3, C), jnp.float32),
            pltpu.VMEM((8 * NUM_LEVELS, C), jnp.int32),
            pltpu.VMEM((8 * NUM_LEVELS, C), jnp.int32),
            pltpu.VMEM((3 * NUM_LEVELS, C), jnp.float32),
            pltpu.VMEM((3 * NUM_LEVELS, C), jnp.float32),
            pltpu.VMEM((8 * NUM_LEVELS, C), jnp.int32),
            pltpu.VMEM((8 * NUM_LEVELS, C), jnp.int32),
            pltpu.VMEM((2 * NUM_LEVELS, C), jnp.float32),
            pltpu.VMEM((2 * NUM_LEVELS, C), jnp.float32),
            pltpu.VMEM((36864,), jnp.int32),
            pltpu.VMEM((16,), jnp.float32),
            pltpu.VMEM((16,), jnp.int32),
            pltpu.VMEM((16,), jnp.int32),
            pltpu.VMEM((16,), jnp.int32),
            pltpu.SemaphoreType.DMA,
            pltpu.SemaphoreType.DMA,
        ],
        compiler_params=pltpu.CompilerParams(needs_layout_passes=False),
    )
    def enc_kernel(pos_hbm, emb_hbm, emb2d_hbm, rm1f_hbm, rm1i_hbm, mask_hbm,
                   off_hbm, enc_hbm, pos0_v, pos1_v, idx0_v, idx1_v,
                   w30_v, w31_v, rows0_v, rows1_v, acc0_v, acc1_v,
                   t01_v, rm1f_v, rm1i_v, mask_v, off_v, sem0, sem1):
        wid = lax.axis_index("s") * 2 + lax.axis_index("c")
        base0 = wid * I32(NP)
        pltpu.sync_copy(rm1f_hbm, rm1f_v)
        pltpu.sync_copy(rm1i_hbm, rm1i_v)
        pltpu.sync_copy(mask_hbm, mask_v)
        pltpu.sync_copy(off_hbm, off_v)
        pltpu.sync_copy(emb_hbm.at[pl.ds(I32(0), 36864)], t01_v)
        b0 = (pos0_v, idx0_v, w30_v, rows0_v, acc0_v, sem0)
        b1 = (pos1_v, idx1_v, w31_v, rows1_v, acc1_v, sem1)

        def phaseA(chunk, buf):
            pos_v, idx_v, w3_v, rows_v, _, sem = buf
            base = base0 + chunk * I32(C)
            pltpu.sync_copy(pos_hbm.at[:, pl.ds(base, C)], pos_v)

            def norm_body(g, _):
                gb = g * I32(16)
                for d in range(3):
                    p = pos_v[d, pl.ds(gb, 16)]
                    p01 = jnp.minimum(jnp.maximum((p + 1.0) * 0.5, 0.0), 1.0)
                    pos_v[d, pl.ds(gb, 16)] = p01
                return 0
            lax.fori_loop(I32(0), I32(G), norm_body, 0, unroll=True)

            def levelA(lv, _):
                lvv = jnp.full((16,), lv, jnp.int32)
                rbase8 = I32(8) * lv - I32(16)
                rm1f_b = plsc.load_gather(rm1f_v, [lvv])
                rm1i_b = plsc.load_gather(rm1i_v, [lvv])
                mask_b = plsc.load_gather(mask_v, [lvv])
                off_b = plsc.load_gather(off_v, [lvv])

                def groupA(g, _):
                    gb = g * I32(16)
                    px = pos_v[0, pl.ds(gb, 16)]
                    py = pos_v[1, pl.ds(gb, 16)]
                    pz = pos_v[2, pl.ds(gb, 16)]
                    psx = px * rm1f_b
                    psy = py * rm1f_b
                    psz = pz * rm1f_b
                    pgx = psx.astype(jnp.int32)
                    pgy = psy.astype(jnp.int32)
                    pgz = psz.astype(jnp.int32)
                    w3_v[I32(3) * lv + I32(0), pl.ds(gb, 16)] = psx - pgx.astype(jnp.float32)
                    w3_v[I32(3) * lv + I32(1), pl.ds(gb, 16)] = psy - pgy.astype(jnp.float32)
                    w3_v[I32(3) * lv + I32(2), pl.ds(gb, 16)] = psz - pgz.astype(jnp.float32)
                    x0 = pgx * HX
                    x1 = jnp.minimum(pgx + I32(1), rm1i_b) * HX
                    y0 = pgy * HY
                    y1 = jnp.minimum(pgy + I32(1), rm1i_b) * HY
                    z0 = pgz * HZ
                    z1 = jnp.minimum(pgz + I32(1), rm1i_b) * HZ
                    xy = (x0 + y0, x0 + y1, x1 + y0, x1 + y1)
                    zz = (z0, z1)
                    for j in range(8):
                        h = (xy[j >> 1] + zz[j & 1]) & mask_b
                        idx_v[rbase8 + I32(j), pl.ds(gb, 16)] = h + off_b
                    return 0
                lax.fori_loop(I32(0), I32(G), groupA, 0, unroll=True)
                for j in range(8):
                    r = rbase8 + I32(j)
                    pltpu.async_copy(emb_hbm.at[idx_v.at[r]], rows_v.at[r], sem)
                return 0
            lax.fori_loop(I32(2), I32(NUM_LEVELS), levelA, 0)

            for lv01 in (0, 1):
                rm1f_c = np.float32(RES[lv01] - 1)
                rm1i_c = I32(RES[lv01] - 1)
                mask_c = I32(MASKS[lv01])
                off_c = I32(OFF[lv01])
                acc_v = buf[4]

                def group01(g, _, lv01=lv01, rm1f_c=rm1f_c, rm1i_c=rm1i_c,
                            mask_c=mask_c, off_c=off_c, acc_v=acc_v,
                            pos_v=pos_v):
                    gb = g * I32(16)
                    px = pos_v[0, pl.ds(gb, 16)]
                    py = pos_v[1, pl.ds(gb, 16)]
                    pz = pos_v[2, pl.ds(gb, 16)]
                    psx = px * rm1f_c
                    psy = py * rm1f_c
                    psz = pz * rm1f_c
                    pgx = psx.astype(jnp.int32)
                    pgy = psy.astype(jnp.int32)
                    pgz = psz.astype(jnp.int32)
                    wx = psx - pgx.astype(jnp.float32)
                    wy = psy - pgy.astype(jnp.float32)
                    wz = psz - pgz.astype(jnp.float32)
                    x0 = pgx * HX
                    x1 = jnp.minimum(pgx + I32(1), rm1i_c) * HX
                    y0 = pgy * HY
                    y1 = jnp.minimum(pgy + I32(1), rm1i_c) * HY
                    z0 = pgz * HZ
                    z1 = jnp.minimum(pgz + I32(1), rm1i_c) * HZ
                    xy = (x0 + y0, x0 + y1, x1 + y0, x1 + y1)
                    zs = (z0, z1)
                    pk = [plsc.load_gather(
                              t01_v,
                              [((xy[j >> 1] + zs[j & 1]) & mask_c) + off_c])
                          for j in range(8)]
                    e0 = [plsc.bitcast(jnp.left_shift(p, I32(16)), jnp.float32)
                          for p in pk]
                    e1 = [plsc.bitcast(p & I32(-65536), jnp.float32)
                          for p in pk]
                    for f, e in ((0, e0), (1, e1)):
                        a00 = e[0] + wx * (e[4] - e[0])
                        a01 = e[1] + wx * (e[5] - e[1])
                        a10 = e[2] + wx * (e[6] - e[2])
                        a11 = e[3] + wx * (e[7] - e[3])
                        b0_ = a00 + wy * (a10 - a00)
                        b1_ = a01 + wy * (a11 - a01)
                        acc_v[I32(2 * lv01 + f), pl.ds(gb, 16)] = \
                            b0_ + wz * (b1_ - b0_)
                    return 0
                lax.fori_loop(I32(0), I32(G), group01, 0, unroll=True)

        def phaseB(chunk, buf):
            _, _, w3_v, rows_v, acc_v, sem = buf
            base = base0 + chunk * I32(C)
            pltpu.make_async_copy(
                emb2d_hbm.at[pl.ds(I32(0), 8 * (NUM_LEVELS - 2)), :],
                rows_v.at[pl.ds(I32(0), 8 * (NUM_LEVELS - 2))], sem).wait()

            def levelB(lv, _):
                rbase8 = I32(8) * lv - I32(16)

                def groupB(g, _):
                    gb = g * I32(16)
                    wx = w3_v[I32(3) * lv + I32(0), pl.ds(gb, 16)]
                    wy = w3_v[I32(3) * lv + I32(1), pl.ds(gb, 16)]
                    wz = w3_v[I32(3) * lv + I32(2), pl.ds(gb, 16)]
                    pk = [rows_v[rbase8 + I32(j), pl.ds(gb, 16)]
                          for j in range(8)]
                    e0 = [plsc.bitcast(jnp.left_shift(p, I32(16)), jnp.float32)
                          for p in pk]
                    e1 = [plsc.bitcast(p & I32(-65536), jnp.float32)
                          for p in pk]
                    for f, e in ((0, e0), (1, e1)):
                        a00 = e[0] + wx * (e[4] - e[0])
                        a01 = e[1] + wx * (e[5] - e[1])
                        a10 = e[2] + wx * (e[6] - e[2])
                        a11 = e[3] + wx * (e[7] - e[3])
                        b0_ = a00 + wy * (a10 - a00)
                        b1_ = a01 + wy * (a11 - a01)
                        acc_v[I32(2) * lv + I32(f), pl.ds(gb, 16)] = b0_ + wz * (b1_ - b0_)
                    return 0
                lax.fori_loop(I32(0), I32(G), groupB, 0, unroll=True)
                return 0
            lax.fori_loop(I32(2), I32(NUM_LEVELS), levelB, 0)

            pltpu.sync_copy(acc_v, enc_hbm.at[:, pl.ds(base, C)])

        phaseA(I32(0), b0)

        def pipe_body(k2, _):
            e = k2 * I32(2)
            phaseA(e + I32(1), b1)
            phaseB(e, b0)
            phaseA(e + I32(2), b0)
            phaseB(e + I32(1), b1)
            return 0
        lax.fori_loop(I32(0), I32(NCHUNK // 2 - 1), pipe_body, 0)

        phaseA(I32(NCHUNK - 1), b1)
        phaseB(I32(NCHUNK - 2), b0)
        phaseB(I32(NCHUNK - 1), b1)

    return enc_kernel(pos_t, emb, emb2d, rm1f, rm1i, maskv, offv)


def _tc_mlp(enc_t, dirs_t, W1T, b1, W2T, b2, Wc1T, bc1, Wc2T, bc2, Wc3T, bc3):
    N = enc_t.shape[1]
    B = 4096

    def body(enc_ref, dirs_ref, W1_ref, b1_ref, W2_ref, b2_ref,
             Wc1_ref, bc1_ref, Wc2_ref, bc2_ref, Wc3_ref, bc3_ref, out_ref,
             c_ref):
        enc = enc_ref[...]
        h1 = jnp.maximum(jnp.dot(W1_ref[...], enc) + b1_ref[...], 0.0)
        h2 = jnp.dot(W2_ref[...], h1) + b2_ref[...]
        d = dirs_ref[...]
        x = d[0:1, :]
        y = d[1:2, :]
        z = d[2:3, :]
        norm = jnp.sqrt(x * x + y * y + z * z)
        x = x / norm
        y = y / norm
        z = z / norm
        xx = x * x
        yy = y * y
        zz = z * z
        c_ref[0:1, :] = jnp.full_like(x, 0.28209479177387814)
        c_ref[1:2, :] = -0.48860251190291987 * y
        c_ref[2:3, :] = 0.48860251190291987 * z
        c_ref[3:4, :] = -0.48860251190291987 * x
        c_ref[4:5, :] = 1.0925484305920792 * x * y
        c_ref[5:6, :] = -1.0925484305920792 * y * z
        c_ref[6:7, :] = 0.31539156525252005 * (2 * zz - xx - yy)
        c_ref[7:8, :] = -1.0925484305920792 * x * z
        c_ref[8:9, :] = 0.5462742152960396 * (xx - yy)
        c_ref[9:10, :] = -0.5900435899266435 * y * (3 * xx - yy)
        c_ref[10:11, :] = 2.890611442640554 * x * y * z
        c_ref[11:12, :] = -0.4570457994644658 * y * (4 * zz - xx - yy)
        c_ref[12:13, :] = 0.3731763325901154 * z * (2 * zz - 3 * xx - 3 * yy)
        c_ref[13:14, :] = -0.4570457994644658 * x * (4 * zz - xx - yy)
        c_ref[14:15, :] = 1.445305721320277 * z * (xx - yy)
        c_ref[15:16, :] = -0.5900435899266435 * x * (xx - 3 * yy)
        c_ref[16:31, :] = h2[1:16, :]
        c_ref[31:32, :] = jnp.zeros_like(x)
        c = c_ref[...]
        c1 = jnp.maximum(jnp.dot(Wc1_ref[...], c) + bc1_ref[...], 0.0)
        c2 = jnp.maximum(jnp.dot(Wc2_ref[...], c1) + bc2_ref[...], 0.0)
        out_ref[0:3, :] = jax.nn.sigmoid(
            jnp.dot(Wc3_ref[...], c2) + bc3_ref[...])
        out_ref[3:4, :] = jnp.exp(h2[0:1, :])

    full = lambda shape: pl.BlockSpec(shape, lambda i: (0, 0))
    return pl.pallas_call(
        body,
        grid=(N // B,),
        in_specs=[
            pl.BlockSpec((2 * NUM_LEVELS, B), lambda i: (0, i)),
            pl.BlockSpec((3, B), lambda i: (0, i)),
            full(W1T.shape), full(b1.shape), full(W2T.shape), full(b2.shape),
            full(Wc1T.shape), full(bc1.shape), full(Wc2T.shape), full(bc2.shape),
            full(Wc3T.shape), full(bc3.shape),
        ],
        out_specs=pl.BlockSpec((4, B), lambda i: (0, i)),
        out_shape=jax.ShapeDtypeStruct((4, N), jnp.float32),
        scratch_shapes=[pltpu.VMEM((32, B), jnp.float32)],
    )(enc_t, dirs_t, W1T, b1, W2T, b2, Wc1T, bc1, Wc2T, bc2, Wc3T, bc3)


def kernel(positions, directions, embeddings, W1, b1, W2, b2,
           Wc1, bc1, Wc2, bc2, Wc3, bc3):
    with jax.enable_x64(False):
        out = _kernel_x32(positions, directions, embeddings, W1, b1, W2, b2,
                          Wc1, bc1, Wc2, bc2, Wc3, bc3)
    return out.astype(jnp.float64)


def _kernel_x32(positions, directions, embeddings, W1, b1, W2, b2,
                Wc1, bc1, Wc2, bc2, Wc3, bc3):
    f32 = jnp.float32
    (positions, directions, embeddings, W1, b1, W2, b2,
     Wc1, bc1, Wc2, bc2, Wc3, bc3) = (
        a.astype(f32) for a in (positions, directions, embeddings, W1, b1,
                                W2, b2, Wc1, bc1, Wc2, bc2, Wc3, bc3))
    pos_t = positions.T
    dirs_t = directions.T
    rm1f = jnp.asarray([r - 1 for r in RES], jnp.float32)
    rm1i = jnp.asarray([r - 1 for r in RES], jnp.int32)
    maskv = jnp.asarray(MASKS, jnp.int32)
    offv = jnp.asarray(OFF, jnp.int32)
    emb_packed = jax.lax.bitcast_convert_type(
        embeddings.astype(jnp.bfloat16), jnp.int32)
    emb2d = emb_packed.reshape(-1, 128)
    N = pos_t.shape[1]
    H = N // 2
    mlp_w = (W1.T, b1.reshape(-1, 1), W2.T, b2.reshape(-1, 1),
             jnp.pad(Wc1.T, ((0, 0), (0, 1))), bc1.reshape(-1, 1),
             Wc2.T, bc2.reshape(-1, 1), Wc3.T, bc3.reshape(-1, 1))
    enc_a = _sc_encode(pos_t[:, :H], emb_packed, emb2d, rm1f, rm1i, maskv, offv)
    enc_b = _sc_encode(pos_t[:, H:], emb_packed, emb2d, rm1f, rm1i, maskv, offv)
    out_a = _tc_mlp(enc_a, dirs_t[:, :H], *mlp_w)
    out_b = _tc_mlp(enc_b, dirs_t[:, H:], *mlp_w)
    out_t = jnp.concatenate([out_a, out_b], axis=1)
    return out_t.T
